# baseline (speedup 1.0000x reference)
import jax
import jax.numpy as jnp
from jax.experimental import pallas as pl

N = 10000
E = 160000
H = 4
OUT = 256


def _tanh_bias_kernel(acc_ref, b_ref, o_ref):
    o_ref[...] = jnp.tanh(acc_ref[...] + b_ref[...])


def kernel(x, edge_index, W1, al1, ar1, b1, W2, al2, ar2, b2, W3, al3, ar3, b3):
    src = edge_index[0]
    dst = edge_index[1]
    feat = (x @ W3).reshape(N, H, OUT)
    el = jnp.sum(feat * al3[None, :, :], axis=-1)
    er = jnp.sum(feat * ar3[None, :, :], axis=-1)
    e = jax.nn.leaky_relu(el[src] + er[dst], negative_slope=0.2)
    m = jax.ops.segment_max(e, dst, num_segments=N)
    m = jnp.where(jnp.isfinite(m), m, 0.0)
    ex = jnp.exp(e - m[dst])
    s = jax.ops.segment_sum(ex, dst, num_segments=N)
    alpha = ex / (s[dst] + 1e-9)
    msg = feat[src] * alpha[..., None]
    acc = jax.ops.segment_sum(msg, dst, num_segments=N).reshape(N, H * OUT)

    out = pl.pallas_call(
        _tanh_bias_kernel,
        grid=(10,),
        in_specs=[
            pl.BlockSpec((N // 10, H * OUT), lambda i: (i, 0)),
            pl.BlockSpec((1, H * OUT), lambda i: (0, 0)),
        ],
        out_specs=pl.BlockSpec((N // 10, H * OUT), lambda i: (i, 0)),
        out_shape=jax.ShapeDtypeStruct((N, H * OUT), jnp.float32),
    )(acc, b3.reshape(1, H * OUT))
    return out.reshape(N, H, OUT)


# speedup vs baseline: 3.8972x; 3.8972x over previous
import functools

import jax
import jax.numpy as jnp
from jax import lax
from jax.experimental import pallas as pl
from jax.experimental.pallas import tpu as pltpu
from jax.experimental.pallas import tpu_sc as plsc

N = 10000
E = 160000
H = 4
IN = 256
OUT = 256
F = H * OUT

NC = 2
NS = 16
NW = NC * NS
TASK_N = 64
NT = (N + TASK_N - 1) // TASK_N
NT_PAD = 160
G = 32
NP = NT * TASK_N


def _mm_kernel(x_ref, w_ref, al_ref, ar_ref, feat_ref, elr_ref):
    xb = x_ref[...]
    f = jnp.dot(xb, w_ref[...], preferred_element_type=jnp.float32)
    feat_ref[...] = f
    f4 = f.reshape(xb.shape[0], H, OUT)
    el = jnp.sum(f4 * al_ref[...][None, :, :], axis=-1)
    er = jnp.sum(f4 * ar_ref[...][None, :, :], axis=-1)
    elr_ref[...] = jnp.concatenate([el, er], axis=1)


def _feat_logits(x, W3, al3, ar3):
    blk = 1000
    return pl.pallas_call(
        _mm_kernel,
        grid=(N // blk,),
        in_specs=[
            pl.BlockSpec((blk, IN), lambda i: (i, 0)),
            pl.BlockSpec((IN, F), lambda i: (0, 0)),
            pl.BlockSpec((H, OUT), lambda i: (0, 0)),
            pl.BlockSpec((H, OUT), lambda i: (0, 0)),
        ],
        out_specs=[
            pl.BlockSpec((blk, F), lambda i: (i, 0)),
            pl.BlockSpec((blk, 2 * H), lambda i: (i, 0)),
        ],
        out_shape=[
            jax.ShapeDtypeStruct((N, F), jnp.float32),
            jax.ShapeDtypeStruct((N, 2 * H), jnp.float32),
        ],
    )(x, W3, al3, ar3)


def _tanh_bias_kernel(acc_ref, b_ref, o_ref):
    o_ref[...] = jnp.tanh(acc_ref[...] + b_ref[...])


def _epilogue(acc, b3):
    blk = 1000
    return pl.pallas_call(
        _tanh_bias_kernel,
        grid=(N // blk,),
        in_specs=[
            pl.BlockSpec((blk, F), lambda i: (i, 0)),
            pl.BlockSpec((1, F), lambda i: (0, 0)),
        ],
        out_specs=pl.BlockSpec((blk, F), lambda i: (i, 0)),
        out_shape=jax.ShapeDtypeStruct((N, F), jnp.float32),
    )(acc, b3.reshape(1, F))


def _sc_agg_body(feat_hbm, src_hbm, dst_hbm, w_hbm, cuts_hbm, out_hbm,
                 acc_v, rows_v, srcidx_v, dst_v, w_v, cuts_v, sem):
    wid = lax.axis_index("s") * NC + lax.axis_index("c")
    pltpu.sync_copy(cuts_hbm, cuts_v)

    def getcut(i):
        return cuts_v[pl.ds(i * 16, 16)][0]

    def task_body(k, carry):
        t = wid + NW * k

        @pl.when(t < NT)
        def _():
            n0 = t * TASK_N

            def zero_row(r, c):
                for cb in range(F // 16):
                    acc_v[r, pl.ds(cb * 16, 16)] = jnp.zeros((16,), jnp.float32)
                return c
            lax.fori_loop(0, TASK_N, zero_row, 0)

            estart = getcut(t)
            eend = getcut(t + 1)
            e0b = (estart // G) * G
            nch = (eend - e0b + (G - 1)) // G

            def chunk(c, cc):
                e0 = e0b + c * G
                pltpu.sync_copy(src_hbm.at[pl.ds(e0, G)], srcidx_v)
                pltpu.sync_copy(dst_hbm.at[pl.ds(e0, G)], dst_v)
                pltpu.sync_copy(w_hbm.at[pl.ds(e0 * H, G * H)], w_v)
                pltpu.async_copy(feat_hbm.at[srcidx_v], rows_v, sem).wait()

                def group(g, ec):
                    gb = g * 16
                    dvec = dst_v[pl.ds(gb, 16)]
                    wvecs = [w_v[pl.ds(g * 16 * H + k * 16, 16)]
                             for k in range(H)]
                    for j in range(16):
                        jg = e0 + gb + j
                        valid = jnp.logical_and(jg >= estart, jg < eend)
                        wscale = jnp.where(valid, 1.0, 0.0).astype(
                            jnp.float32)
                        dl = jnp.clip(dvec[j] - n0, 0, TASK_N - 1)
                        r = gb + j
                        ws = [wvecs[(j * H + h) // 16][(j * H + h) % 16]
                              * wscale for h in range(H)]

                        def fblk(fb, fc):
                            off = fb * 16
                            for h in range(H):
                                o = h * OUT + off
                                v = rows_v[r, pl.ds(o, 16)]
                                acc_v[dl, pl.ds(o, 16)] += v * ws[h]
                            return fc
                        lax.fori_loop(0, OUT // 16, fblk, 0)
                    return ec
                lax.fori_loop(0, G // 16, group, 0)
                return cc
            lax.fori_loop(0, nch, chunk, 0)
            pltpu.sync_copy(acc_v, out_hbm.at[pl.ds(n0, TASK_N)])
        return carry

    lax.fori_loop(0, (NT + NW - 1) // NW, task_body, 0)


def _sc_aggregate(feat, src_p, dst_p, w_flat, cuts):
    mesh = plsc.VectorSubcoreMesh(core_axis_name="c", subcore_axis_name="s")
    run = functools.partial(
        pl.kernel,
        mesh=mesh,
        out_type=jax.ShapeDtypeStruct((NP, F), jnp.float32),
        scratch_types=[
            pltpu.VMEM((TASK_N, F), jnp.float32),
            pltpu.VMEM((G, F), jnp.float32),
            pltpu.VMEM((G,), jnp.int32),
            pltpu.VMEM((G,), jnp.int32),
            pltpu.VMEM((G * H,), jnp.float32),
            pltpu.VMEM((NT_PAD * 16,), jnp.int32),
            pltpu.SemaphoreType.DMA,
        ],
    )(_sc_agg_body)
    return run(feat, src_p, dst_p, w_flat, cuts)


def kernel(x, edge_index, W1, al1, ar1, b1, W2, al2, ar2, b2, W3, al3, ar3, b3):
    src = edge_index[0]
    dst = edge_index[1]

    feat, elr = _feat_logits(x, W3, al3, ar3)

    dst_s, src_s = lax.sort_key_val(dst, src)
    cuts = jnp.searchsorted(
        dst_s, jnp.arange(NT, dtype=jnp.int32) * TASK_N, side="left"
    ).astype(jnp.int32)
    cuts = jnp.concatenate(
        [cuts, jnp.full((NT_PAD - NT,), E, jnp.int32)])
    cuts_exp = jnp.zeros((NT_PAD * 16,), jnp.int32)
    cuts_exp = cuts_exp.at[jnp.arange(NT_PAD) * 16].set(cuts)

    el = elr[:, :H]
    er = elr[:, H:]
    e = jax.nn.leaky_relu(el[src_s] + er[dst_s], negative_slope=0.2)
    m = jax.ops.segment_max(e, dst_s, num_segments=N)
    m = jnp.where(jnp.isfinite(m), m, 0.0)
    ex = jnp.exp(e - m[dst_s])
    s = jax.ops.segment_sum(ex, dst_s, num_segments=N)
    alpha = ex / (s[dst_s] + 1e-9)

    src_p = jnp.concatenate([src_s, jnp.zeros((G,), jnp.int32)])
    dst_p = jnp.concatenate([dst_s, jnp.zeros((G,), jnp.int32)])
    w_flat = jnp.concatenate(
        [alpha.reshape(-1), jnp.zeros((G * H,), jnp.float32)])

    acc = _sc_aggregate(feat, src_p, dst_p, w_flat, cuts_exp)[:N]
    out = _epilogue(acc, b3)
    return out.reshape(N, H, OUT)
